# baseline (speedup 1.0000x reference)
import jax
import jax.numpy as jnp
from jax import lax
from jax.experimental import pallas as pl
from jax.experimental.pallas import tpu as pltpu
from jax.experimental.pallas import tpu_sc as plsc

_NC = 2
_NS = 16
_NW = _NC * _NS

_B, _L = 4096, 200
_N = _B * _L
_V = 21
_D = 21
_DP = 24
_IW = 128
_PER_W = _N // _NW
_TILES_W = _PER_W // _IW
_G = 20
_NGRP = _TILES_W // _G


def _body(idx_hbm, table_hbm, out_hbm, idx_all, sem_g, sem_o, *row_bufs):
    wid = lax.axis_index("s") * _NC + lax.axis_index("c")
    pltpu.sync_copy(idx_hbm.at[wid], idx_all)

    def step(i, carry):
        t0 = i * _G
        gathers = [
            pltpu.async_copy(table_hbm.at[idx_all.at[t0 + j]],
                             row_bufs[j], sem_g)
            for j in range(_G)
        ]
        for g in gathers:
            g.wait()
        base = wid * _PER_W + t0 * _IW
        writes = [
            pltpu.async_copy(row_bufs[j],
                             out_hbm.at[pl.ds(base + j * _IW, _IW)], sem_o)
            for j in range(_G)
        ]
        for w in writes:
            w.wait()
        return carry

    lax.fori_loop(0, _NGRP, step, 0)


def kernel(sequence_indices, table):
    idx_rows = sequence_indices.reshape(_NW, _TILES_W, _IW)
    table_padded = jnp.pad(table, ((0, 0), (0, _DP - _D)))
    mesh = plsc.VectorSubcoreMesh(
        core_axis_name="c", subcore_axis_name="s",
        num_cores=_NC, num_subcores=_NS,
    )
    k = pl.kernel(
        _body,
        out_type=jax.ShapeDtypeStruct((_N, _DP), jnp.float32),
        mesh=mesh,
        scratch_types=[
            pltpu.VMEM((_TILES_W, _IW), jnp.int32),
            pltpu.SemaphoreType.DMA,
            pltpu.SemaphoreType.DMA,
        ] + [pltpu.VMEM((_IW, _DP), jnp.float32) for _ in range(_G)],
        compiler_params=pltpu.CompilerParams(use_tc_tiling_on_sc=False),
    )
    out = k(idx_rows, table_padded)
    return out[:, :_D].reshape(_B, _L, _D)


# speedup vs baseline: 2.4755x; 2.4755x over previous
import jax
import jax.numpy as jnp
from jax import lax
from jax.experimental import pallas as pl
from jax.experimental.pallas import tpu as pltpu
from jax.experimental.pallas import tpu_sc as plsc

_NC = 2
_NS = 16
_NW = _NC * _NS

_B, _L = 4096, 200
_N = _B * _L
_V = 21
_D = 21
_DP = 24
_IW = 128
_PER_W = _N // _NW
_TILES_W = _PER_W // _IW
_G = 20
_NGRP = _TILES_W // _G
_R = 512


def _body(idx_hbm, table_hbm, out_hbm, idx_all, sem_g, sem_o, *row_bufs):
    wid = lax.axis_index("s") * _NC + lax.axis_index("c")
    pltpu.sync_copy(idx_hbm.at[wid], idx_all)

    def step(i, carry):
        t0 = i * _G
        gathers = [
            pltpu.async_copy(table_hbm.at[idx_all.at[t0 + j]],
                             row_bufs[j], sem_g)
            for j in range(_G)
        ]
        for g in gathers:
            g.wait()
        base = wid * _PER_W + t0 * _IW
        writes = [
            pltpu.async_copy(row_bufs[j],
                             out_hbm.at[pl.ds(base + j * _IW, _IW)], sem_o)
            for j in range(_G)
        ]
        for w in writes:
            w.wait()
        return carry

    lax.fori_loop(0, _NGRP, step, 0)


def kernel(sequence_indices, table):
    rep_off = _V * (jnp.arange(_N, dtype=jnp.int32) % _R)
    idx_rows = (sequence_indices.reshape(_N) + rep_off).reshape(
        _NW, _TILES_W, _IW)
    table_padded = jnp.tile(jnp.pad(table, ((0, 0), (0, _DP - _D))),
                            (_R, 1))
    mesh = plsc.VectorSubcoreMesh(
        core_axis_name="c", subcore_axis_name="s",
        num_cores=_NC, num_subcores=_NS,
    )
    k = pl.kernel(
        _body,
        out_type=jax.ShapeDtypeStruct((_N, _DP), jnp.float32),
        mesh=mesh,
        scratch_types=[
            pltpu.VMEM((_TILES_W, _IW), jnp.int32),
            pltpu.SemaphoreType.DMA,
            pltpu.SemaphoreType.DMA,
        ] + [pltpu.VMEM((_IW, _DP), jnp.float32) for _ in range(_G)],
        compiler_params=pltpu.CompilerParams(use_tc_tiling_on_sc=False),
    )
    out = k(idx_rows, table_padded)
    return out[:, :_D].reshape(_B, _L, _D)
